# baseline (speedup 1.0000x reference)
import functools

import jax
import jax.numpy as jnp
from jax import lax
from jax.experimental import pallas as pl
from jax.experimental.pallas import tpu as pltpu
from jax.experimental.pallas import tpu_sc as plsc

DIM = 64
LANES = 16
NUM_WORKERS = 32
CHUNK = 64
DCHUNK = 80


def _sc_depad(E_w):
    V = E_w.shape[0]
    n_chunks = V // DCHUNK
    mesh = plsc.VectorSubcoreMesh(core_axis_name="c", subcore_axis_name="s")

    in_t = pltpu.VMEM((DCHUNK, DIM), jnp.float32)
    out_t = pltpu.VMEM((DCHUNK // 2, 2 * DIM), jnp.float32)

    @functools.partial(
        pl.kernel,
        mesh=mesh,
        out_type=jax.ShapeDtypeStruct((V // 2, 2 * DIM), jnp.float32),
        scratch_types=[
            in_t, in_t, out_t,
            pltpu.SemaphoreType.DMA,
        ],
    )
    def k(E_h, out_h, ibuf0, ibuf1, obuf, sem_in):
        wid = lax.axis_index("s") * 2 + lax.axis_index("c")
        ibufs = (ibuf0, ibuf1)
        full_iters = n_chunks // NUM_WORKERS
        tail_workers = n_chunks % NUM_WORKERS

        def fire_in(i, b):
            c = wid + i * NUM_WORKERS
            pltpu.async_copy(E_h.at[pl.ds(c * DCHUNK, DCHUNK)], ibufs[b],
                             sem_in)

        def wait_in(i, b):
            c = wid + i * NUM_WORKERS
            pltpu.make_async_copy(E_h.at[pl.ds(c * DCHUNK, DCHUNK)], ibufs[b],
                                  sem_in).wait()

        def repack(b):
            ib = ibufs[b]

            @plsc.parallel_loop(0, DCHUNK // 2, unroll=4)
            def _(pr):
                r = 2 * pr
                for kk in range(4):
                    obuf[pr, pl.ds(kk * 16, 16)] = ib[r, pl.ds(kk * 16, 16)]
                for kk in range(4):
                    obuf[pr, pl.ds(DIM + kk * 16, 16)] = ib[r + 1,
                                                            pl.ds(kk * 16, 16)]

        def copy_out(i):
            c = wid + i * NUM_WORKERS
            pltpu.sync_copy(obuf,
                            out_h.at[pl.ds(c * (DCHUNK // 2), DCHUNK // 2)])

        fire_in(0, 0)

        def pair(p, _):
            i0 = 2 * p
            wait_in(i0, 0)
            fire_in(i0 + 1, 1)
            repack(0)
            copy_out(i0)
            wait_in(i0 + 1, 1)

            @pl.when(i0 + 2 < full_iters)
            def _():
                fire_in(i0 + 2, 0)

            @pl.when(jnp.logical_and(i0 + 2 == full_iters,
                                     wid < tail_workers))
            def _():
                fire_in(i0 + 2, 0)

            repack(1)
            copy_out(i0 + 1)
            return 0

        lax.fori_loop(0, full_iters // 2, pair, 0)

        @pl.when(wid < tail_workers)
        def _():
            wait_in(full_iters, 0)
            repack(0)
            copy_out(full_iters)

    return k(E_w)


def _sc_score(srow, soff, rrow, roff, orow, ooff, trow, toff, E2, R2, T2):
    total = srow.shape[0]
    per_w = total // NUM_WORKERS
    n_chunks = per_w // CHUNK

    mesh = plsc.VectorSubcoreMesh(core_axis_name="c", subcore_axis_name="s")

    idx_t = pltpu.VMEM((per_w,), jnp.int32)
    row_buf = pltpu.VMEM((2, CHUNK, 2 * DIM), jnp.float32)

    @functools.partial(
        pl.kernel,
        mesh=mesh,
        out_type=jax.ShapeDtypeStruct((total,), jnp.float32),
        scratch_types=[
            idx_t, idx_t, idx_t, idx_t,
            idx_t, idx_t, idx_t, idx_t,
            row_buf, row_buf, row_buf, row_buf,
            pltpu.VMEM((per_w,), jnp.float32),
            pltpu.SemaphoreType.DMA,
            pltpu.SemaphoreType.DMA,
        ],
    )
    def k(srow_h, soff_h, rrow_h, roff_h, orow_h, ooff_h, trow_h, toff_h,
          E_h, R_h, T_h, out_h,
          sidx, ridx, oidx, tidx, soffv, roffv, ooffv, toffv,
          srows, rrows, orows, trows, outv, sem0, sem1):
        wid = lax.axis_index("s") * 2 + lax.axis_index("c")
        base = wid * per_w

        pltpu.sync_copy(srow_h.at[pl.ds(base, per_w)], sidx)
        pltpu.sync_copy(rrow_h.at[pl.ds(base, per_w)], ridx)
        pltpu.sync_copy(orow_h.at[pl.ds(base, per_w)], oidx)
        pltpu.sync_copy(trow_h.at[pl.ds(base, per_w)], tidx)
        pltpu.sync_copy(soff_h.at[pl.ds(base, per_w)], soffv)
        pltpu.sync_copy(roff_h.at[pl.ds(base, per_w)], roffv)
        pltpu.sync_copy(ooff_h.at[pl.ds(base, per_w)], ooffv)
        pltpu.sync_copy(toff_h.at[pl.ds(base, per_w)], toffv)

        def fire(ci, b, sem):
            sl = pl.ds(ci * CHUNK, CHUNK)
            pltpu.async_copy(E_h.at[sidx.at[sl]], srows.at[b], sem)
            pltpu.async_copy(R_h.at[ridx.at[sl]], rrows.at[b], sem)
            pltpu.async_copy(E_h.at[oidx.at[sl]], orows.at[b], sem)
            pltpu.async_copy(T_h.at[tidx.at[sl]], trows.at[b], sem)

        def drain(ci, b, sem):
            sl = pl.ds(ci * CHUNK, CHUNK)
            pltpu.make_async_copy(E_h.at[sidx.at[sl]], srows.at[b], sem).wait()
            pltpu.make_async_copy(R_h.at[ridx.at[sl]], rrows.at[b], sem).wait()
            pltpu.make_async_copy(E_h.at[oidx.at[sl]], orows.at[b], sem).wait()
            pltpu.make_async_copy(T_h.at[tidx.at[sl]], trows.at[b], sem).wait()

        lane = lax.iota(jnp.int32, LANES)

        def allsum(v):
            for sh in (8, 4, 2, 1):
                v = v + v.at[lane ^ sh].get(mode="promise_in_bounds")
            return v

        def compute(ci, b):
            sb, rb, ob, tb = srows.at[b], rrows.at[b], orows.at[b], trows.at[b]

            def group(g, _):
                eb = g * LANES
                ga = ci * CHUNK + eb
                soffs = soffv[pl.ds(ga, LANES)]
                roffs = roffv[pl.ds(ga, LANES)]
                ooffs = ooffv[pl.ds(ga, LANES)]
                toffs = toffv[pl.ds(ga, LANES)]
                acc = jnp.zeros((LANES,), jnp.float32)
                for j in range(LANES):
                    e = eb + j
                    so = soffs[j]
                    ro = roffs[j]
                    oo = ooffs[j]
                    to = toffs[j]
                    x0 = (sb[e, pl.ds(so, LANES)] + rb[e, pl.ds(ro, LANES)]
                          - ob[e, pl.ds(oo, LANES)])
                    x1 = (sb[e, pl.ds(so + 16, LANES)]
                          + rb[e, pl.ds(ro + 16, LANES)]
                          - ob[e, pl.ds(oo + 16, LANES)])
                    x2 = (sb[e, pl.ds(so + 32, LANES)]
                          + rb[e, pl.ds(ro + 32, LANES)]
                          - ob[e, pl.ds(oo + 32, LANES)])
                    x3 = (sb[e, pl.ds(so + 48, LANES)]
                          + rb[e, pl.ds(ro + 48, LANES)]
                          - ob[e, pl.ds(oo + 48, LANES)])
                    t0 = tb[e, pl.ds(to, LANES)]
                    t1 = tb[e, pl.ds(to + 16, LANES)]
                    t2 = tb[e, pl.ds(to + 32, LANES)]
                    t3 = tb[e, pl.ds(to + 48, LANES)]
                    p = (x0 * t0 + x1 * t1) + (x2 * t2 + x3 * t3)
                    inner = allsum(p)
                    a = (jnp.abs(x0 - t0 * inner) + jnp.abs(x1 - t1 * inner)
                         + jnp.abs(x2 - t2 * inner) + jnp.abs(x3 - t3 * inner))
                    acc = jnp.where(lane == j, allsum(a), acc)
                outv[pl.ds(ci * CHUNK + eb, LANES)] = acc
                return 0

            lax.fori_loop(0, CHUNK // LANES, group, 0)

        fire(0, 0, sem0)

        def pair(p, _):
            c0 = 2 * p
            fire(c0 + 1, 1, sem1)
            drain(c0, 0, sem0)
            compute(c0, 0)

            @pl.when(c0 + 2 < n_chunks)
            def _():
                fire(c0 + 2, 0, sem0)

            drain(c0 + 1, 1, sem1)
            compute(c0 + 1, 1)
            return 0

        lax.fori_loop(0, n_chunks // 2, pair, 0)
        pltpu.sync_copy(outv, out_h.at[pl.ds(base, per_w)])

    return k(srow, soff, rrow, roff, orow, ooff, trow, toff, E2, R2, T2)


def kernel(s, r, o, t, E_w, R_w, T_w):
    B, N = s.shape
    s_f = s.reshape(-1)
    r_f = r.reshape(-1)
    o_f = o.reshape(-1)
    t_f = t[:, :, 0].reshape(-1)
    E2 = _sc_depad(E_w)
    R2 = R_w.reshape(-1, 2 * DIM)
    T2 = jnp.pad(T_w, ((0, 1), (0, 0))).reshape(-1, 2 * DIM)
    out = _sc_score(
        s_f >> 1, (s_f & 1) << 6,
        r_f >> 1, (r_f & 1) << 6,
        o_f >> 1, (o_f & 1) << 6,
        t_f >> 1, (t_f & 1) << 6,
        E2, R2, T2)
    return out.reshape(B, N)


# speedup vs baseline: 1.4292x; 1.4292x over previous
import functools

import jax
import jax.numpy as jnp
from jax import lax
from jax.experimental import pallas as pl
from jax.experimental.pallas import tpu as pltpu
from jax.experimental.pallas import tpu_sc as plsc

DIM = 64
LANES = 16
NUM_WORKERS = 32
CHUNK = 128


def _sc_score(s_f, r_f, o_f, t_f, E_w, R_w, T_w):
    total = s_f.shape[0]
    per_w = total // NUM_WORKERS
    n_chunks = per_w // CHUNK

    mesh = plsc.VectorSubcoreMesh(core_axis_name="c", subcore_axis_name="s")

    row_buf = pltpu.VMEM((2, CHUNK, DIM), jnp.float32)

    @functools.partial(
        pl.kernel,
        mesh=mesh,
        out_type=jax.ShapeDtypeStruct((total,), jnp.float32),
        compiler_params=pltpu.CompilerParams(use_tc_tiling_on_sc=False),
        scratch_types=[
            pltpu.VMEM((per_w,), jnp.int32),
            pltpu.VMEM((per_w,), jnp.int32),
            pltpu.VMEM((per_w,), jnp.int32),
            pltpu.VMEM((per_w,), jnp.int32),
            row_buf, row_buf, row_buf, row_buf,
            pltpu.VMEM((per_w,), jnp.float32),
            pltpu.SemaphoreType.DMA,
            pltpu.SemaphoreType.DMA,
        ],
    )
    def k(s_hbm, r_hbm, o_hbm, t_hbm, E_hbm, R_hbm, T_hbm, out_hbm,
          sidx, ridx, oidx, tidx, srows, rrows, orows, trows, outv,
          sem0, sem1):
        wid = lax.axis_index("s") * 2 + lax.axis_index("c")
        base = wid * per_w

        pltpu.sync_copy(s_hbm.at[pl.ds(base, per_w)], sidx)
        pltpu.sync_copy(r_hbm.at[pl.ds(base, per_w)], ridx)
        pltpu.sync_copy(o_hbm.at[pl.ds(base, per_w)], oidx)
        pltpu.sync_copy(t_hbm.at[pl.ds(base, per_w)], tidx)

        def fire(ci, b, sem):
            sl = pl.ds(ci * CHUNK, CHUNK)
            pltpu.async_copy(E_hbm.at[sidx.at[sl]], srows.at[b], sem)
            pltpu.async_copy(R_hbm.at[ridx.at[sl]], rrows.at[b], sem)
            pltpu.async_copy(E_hbm.at[oidx.at[sl]], orows.at[b], sem)
            pltpu.async_copy(T_hbm.at[tidx.at[sl]], trows.at[b], sem)

        def drain(ci, b, sem):
            sl = pl.ds(ci * CHUNK, CHUNK)
            pltpu.make_async_copy(E_hbm.at[sidx.at[sl]], srows.at[b], sem).wait()
            pltpu.make_async_copy(R_hbm.at[ridx.at[sl]], rrows.at[b], sem).wait()
            pltpu.make_async_copy(E_hbm.at[oidx.at[sl]], orows.at[b], sem).wait()
            pltpu.make_async_copy(T_hbm.at[tidx.at[sl]], trows.at[b], sem).wait()

        lane = lax.iota(jnp.int32, LANES)

        def allsum(v):
            for sh in (8, 4, 2, 1):
                v = v + v.at[lane ^ sh].get(mode="promise_in_bounds")
            return v

        def compute(ci, b):
            sb, rb, ob, tb = srows.at[b], rrows.at[b], orows.at[b], trows.at[b]

            def group(g, _):
                eb = g * LANES
                acc = jnp.zeros((LANES,), jnp.float32)
                for j in range(LANES):
                    e = eb + j
                    x0 = (sb[e, pl.ds(0, LANES)] + rb[e, pl.ds(0, LANES)]
                          - ob[e, pl.ds(0, LANES)])
                    x1 = (sb[e, pl.ds(16, LANES)] + rb[e, pl.ds(16, LANES)]
                          - ob[e, pl.ds(16, LANES)])
                    x2 = (sb[e, pl.ds(32, LANES)] + rb[e, pl.ds(32, LANES)]
                          - ob[e, pl.ds(32, LANES)])
                    x3 = (sb[e, pl.ds(48, LANES)] + rb[e, pl.ds(48, LANES)]
                          - ob[e, pl.ds(48, LANES)])
                    t0 = tb[e, pl.ds(0, LANES)]
                    t1 = tb[e, pl.ds(16, LANES)]
                    t2 = tb[e, pl.ds(32, LANES)]
                    t3 = tb[e, pl.ds(48, LANES)]
                    p = (x0 * t0 + x1 * t1) + (x2 * t2 + x3 * t3)
                    inner = allsum(p)
                    a = (jnp.abs(x0 - t0 * inner) + jnp.abs(x1 - t1 * inner)
                         + jnp.abs(x2 - t2 * inner) + jnp.abs(x3 - t3 * inner))
                    acc = jnp.where(lane == j, allsum(a), acc)
                outv[pl.ds(ci * CHUNK + eb, LANES)] = acc
                return 0

            lax.fori_loop(0, CHUNK // LANES, group, 0)

        fire(0, 0, sem0)

        def pair(p, _):
            c0 = 2 * p
            fire(c0 + 1, 1, sem1)
            drain(c0, 0, sem0)
            compute(c0, 0)

            @pl.when(c0 + 2 < n_chunks)
            def _():
                fire(c0 + 2, 0, sem0)

            drain(c0 + 1, 1, sem1)
            compute(c0 + 1, 1)
            return 0

        lax.fori_loop(0, n_chunks // 2, pair, 0)
        pltpu.sync_copy(outv, out_hbm.at[pl.ds(base, per_w)])

    return k(s_f, r_f, o_f, t_f, E_w, R_w, T_w)


def kernel(s, r, o, t, E_w, R_w, T_w):
    B, N = s.shape
    t_idx = t[:, :, 0].reshape(-1)
    out = _sc_score(s.reshape(-1), r.reshape(-1), o.reshape(-1), t_idx,
                    E_w, R_w, T_w)
    return out.reshape(B, N)


# speedup vs baseline: 1.4957x; 1.0465x over previous
import functools

import jax
import jax.numpy as jnp
from jax import lax
from jax.experimental import pallas as pl
from jax.experimental.pallas import tpu as pltpu
from jax.experimental.pallas import tpu_sc as plsc

DIM = 64
LANES = 16
NUM_WORKERS = 32
CHUNK = 128


def _sc_score(s_f, r_f, o_f, t_f, E_w, R_w, T_w):
    total = s_f.shape[0]
    per_w = total // NUM_WORKERS
    n_chunks = per_w // CHUNK

    mesh = plsc.VectorSubcoreMesh(core_axis_name="c", subcore_axis_name="s")

    row_buf = pltpu.VMEM((2, CHUNK, DIM), jnp.float32)

    @functools.partial(
        pl.kernel,
        mesh=mesh,
        out_type=jax.ShapeDtypeStruct((total,), jnp.float32),
        compiler_params=pltpu.CompilerParams(use_tc_tiling_on_sc=False),
        scratch_types=[
            pltpu.VMEM((per_w,), jnp.int32),
            pltpu.VMEM((per_w,), jnp.int32),
            pltpu.VMEM((per_w,), jnp.int32),
            pltpu.VMEM((per_w,), jnp.int32),
            row_buf, row_buf, row_buf,
            pltpu.VMEM((365, DIM), jnp.float32),
            pltpu.VMEM((per_w,), jnp.float32),
            pltpu.SemaphoreType.DMA,
            pltpu.SemaphoreType.DMA,
        ],
    )
    def k(s_hbm, r_hbm, o_hbm, t_hbm, E_hbm, R_hbm, T_hbm, out_hbm,
          sidx, ridx, oidx, tidx, srows, rrows, orows, T_v, outv,
          sem0, sem1):
        wid = lax.axis_index("s") * 2 + lax.axis_index("c")
        base = wid * per_w

        pltpu.sync_copy(s_hbm.at[pl.ds(base, per_w)], sidx)
        pltpu.sync_copy(r_hbm.at[pl.ds(base, per_w)], ridx)
        pltpu.sync_copy(o_hbm.at[pl.ds(base, per_w)], oidx)
        pltpu.sync_copy(t_hbm.at[pl.ds(base, per_w)], tidx)
        pltpu.sync_copy(T_hbm, T_v)

        def fire(ci, b, sem):
            sl = pl.ds(ci * CHUNK, CHUNK)
            pltpu.async_copy(E_hbm.at[sidx.at[sl]], srows.at[b], sem)
            pltpu.async_copy(R_hbm.at[ridx.at[sl]], rrows.at[b], sem)
            pltpu.async_copy(E_hbm.at[oidx.at[sl]], orows.at[b], sem)

        def drain(ci, b, sem):
            sl = pl.ds(ci * CHUNK, CHUNK)
            pltpu.make_async_copy(E_hbm.at[sidx.at[sl]], srows.at[b], sem).wait()
            pltpu.make_async_copy(R_hbm.at[ridx.at[sl]], rrows.at[b], sem).wait()
            pltpu.make_async_copy(E_hbm.at[oidx.at[sl]], orows.at[b], sem).wait()

        lane = lax.iota(jnp.int32, LANES)

        def allsum(v):
            for sh in (8, 4, 2, 1):
                v = v + v.at[lane ^ sh].get(mode="promise_in_bounds")
            return v

        def compute(ci, b):
            sb, rb, ob = srows.at[b], rrows.at[b], orows.at[b]

            def group(g, _):
                eb = g * LANES
                trows16 = tidx[pl.ds(ci * CHUNK + eb, LANES)]
                acc = jnp.zeros((LANES,), jnp.float32)
                for j in range(LANES):
                    e = eb + j
                    tr = trows16[j]
                    x0 = (sb[e, pl.ds(0, LANES)] + rb[e, pl.ds(0, LANES)]
                          - ob[e, pl.ds(0, LANES)])
                    x1 = (sb[e, pl.ds(16, LANES)] + rb[e, pl.ds(16, LANES)]
                          - ob[e, pl.ds(16, LANES)])
                    x2 = (sb[e, pl.ds(32, LANES)] + rb[e, pl.ds(32, LANES)]
                          - ob[e, pl.ds(32, LANES)])
                    x3 = (sb[e, pl.ds(48, LANES)] + rb[e, pl.ds(48, LANES)]
                          - ob[e, pl.ds(48, LANES)])
                    t0 = T_v[tr, pl.ds(0, LANES)]
                    t1 = T_v[tr, pl.ds(16, LANES)]
                    t2 = T_v[tr, pl.ds(32, LANES)]
                    t3 = T_v[tr, pl.ds(48, LANES)]
                    p = (x0 * t0 + x1 * t1) + (x2 * t2 + x3 * t3)
                    inner = allsum(p)
                    a = (jnp.abs(x0 - t0 * inner) + jnp.abs(x1 - t1 * inner)
                         + jnp.abs(x2 - t2 * inner) + jnp.abs(x3 - t3 * inner))
                    acc = jnp.where(lane == j, allsum(a), acc)
                outv[pl.ds(ci * CHUNK + eb, LANES)] = acc
                return 0

            lax.fori_loop(0, CHUNK // LANES, group, 0)

        fire(0, 0, sem0)

        def pair(p, _):
            c0 = 2 * p
            fire(c0 + 1, 1, sem1)
            drain(c0, 0, sem0)
            compute(c0, 0)

            @pl.when(c0 + 2 < n_chunks)
            def _():
                fire(c0 + 2, 0, sem0)

            drain(c0 + 1, 1, sem1)
            compute(c0 + 1, 1)
            return 0

        lax.fori_loop(0, n_chunks // 2, pair, 0)
        pltpu.sync_copy(outv, out_hbm.at[pl.ds(base, per_w)])

    return k(s_f, r_f, o_f, t_f, E_w, R_w, T_w)


def kernel(s, r, o, t, E_w, R_w, T_w):
    B, N = s.shape
    t_idx = t[:, :, 0].reshape(-1)
    out = _sc_score(s.reshape(-1), r.reshape(-1), o.reshape(-1), t_idx,
                    E_w, R_w, T_w)
    return out.reshape(B, N)
